# baseline (speedup 1.0000x reference)
import functools

import jax
import jax.numpy as jnp
from jax import lax
from jax.experimental import pallas as pl
from jax.experimental.pallas import tpu as pltpu
from jax.experimental.pallas import tpu_sc as plsc

N = 10000
F = 128
TM = 200
NB = N // TM

SC_ROWS = 1024
NWORK = 32
RPW = SC_ROWS // NWORK
NL = 16


def _sc_merge_body(a_ref, wb_ref, out_ref, b0_ref, b1_ref, m_ref, wbv_ref):
    wid = lax.axis_index("s") * 2 + lax.axis_index("c")
    pltpu.sync_copy(wb_ref, wbv_ref)
    v0 = wbv_ref[0, :]
    v1 = wbv_ref[1, :]

    def row_step(k, _):
        row = wid * RPW + k
        pltpu.sync_copy(a_ref.at[0, row], b0_ref)
        pltpu.sync_copy(a_ref.at[1, row], b1_ref)

        def col_step(j, _):
            sl = pl.ds(j * NL, NL)
            m_ref[sl] = b0_ref[sl] * v0 + b1_ref[sl] * v1
            return ()

        lax.fori_loop(0, N // NL, col_step, ())
        pltpu.sync_copy(m_ref, out_ref.at[row])
        return ()

    lax.fori_loop(0, RPW, row_step, ())


def _sc_merge(A, weight_b):
    wbv = jnp.broadcast_to(weight_b.reshape(2, 1), (2, NL)).astype(jnp.float32)
    mesh = plsc.VectorSubcoreMesh(core_axis_name="c", subcore_axis_name="s")
    k = functools.partial(
        pl.kernel,
        out_type=jax.ShapeDtypeStruct((SC_ROWS, N), jnp.float32),
        mesh=mesh,
        scratch_types=[
            pltpu.VMEM((N,), jnp.float32),
            pltpu.VMEM((N,), jnp.float32),
            pltpu.VMEM((N,), jnp.float32),
            pltpu.VMEM((2, NL), jnp.float32),
        ],
    )(_sc_merge_body)
    return k(A, wbv)


def _body(wb_ref, wa_ref, a0_ref, a1_ref, f_ref, w1_ref, b1_ref, w2_ref,
          b2_ref, fa8_ref, o_ref,
          s1_ref, u1_ref, s2_ref, spill_ref, fetch_ref, sem_out, sem_in):
    i = pl.program_id(0)

    @pl.when(i == 0)
    def _():
        s1_ref[...] = jnp.dot(f_ref[...], w1_ref[...],
                              preferred_element_type=jnp.float32
                              ).astype(jnp.bfloat16)

    @pl.when(i < NB)
    def _phase1():
        slot = jax.lax.rem(i, 2)

        @pl.when(i >= 2)
        def _():
            pltpu.make_async_copy(
                spill_ref.at[slot],
                fa8_ref.at[pl.ds((i - 2) * TM, TM), :],
                sem_out.at[slot]).wait()

        wb0 = wb_ref[0, 0]
        wb1 = wb_ref[1, 0]
        m = a0_ref[0] * wb0 + a1_ref[0] * wb1
        spill_ref[slot] = m.astype(jnp.float8_e4m3fn)
        pltpu.make_async_copy(
            spill_ref.at[slot],
            fa8_ref.at[pl.ds(i * TM, TM), :],
            sem_out.at[slot]).start()
        mb = m.astype(jnp.bfloat16)
        u1 = jnp.dot(mb, s1_ref[...], preferred_element_type=jnp.float32)
        u1 = u1 + b1_ref[...]
        u1_ref[pl.ds(i * TM, TM), :] = u1
        s2_ref[pl.ds(i * TM, TM), :] = jnp.dot(
            u1.astype(jnp.bfloat16), w2_ref[...],
            preferred_element_type=jnp.float32).astype(jnp.float8_e4m3fn)

    @pl.when(i >= NB)
    def _phase2():
        j = i - NB
        slot = jax.lax.rem(j, 2)

        @pl.when(j < 2)
        def _():
            pltpu.make_async_copy(
                spill_ref.at[slot],
                fa8_ref.at[pl.ds((NB - 2 + j) * TM, TM), :],
                sem_out.at[slot]).wait()

        @pl.when(j == 0)
        def _():
            pltpu.make_async_copy(
                fa8_ref.at[pl.ds(0, TM), :],
                fetch_ref.at[0],
                sem_in.at[0]).start()

        @pl.when(j < NB - 1)
        def _():
            nxt = jax.lax.rem(j + 1, 2)
            pltpu.make_async_copy(
                fa8_ref.at[pl.ds((j + 1) * TM, TM), :],
                fetch_ref.at[nxt],
                sem_in.at[nxt]).start()

        pltpu.make_async_copy(
            fa8_ref.at[pl.ds(j * TM, TM), :],
            fetch_ref.at[slot],
            sem_in.at[slot]).wait()

        wa = wa_ref[0, 0]
        u2 = jnp.dot(fetch_ref[slot], s2_ref[...],
                     preferred_element_type=jnp.float32)
        u2 = u2 + b2_ref[...]
        o_ref[...] = (u1_ref[pl.ds(j * TM, TM), :] + u2 * wa) * 0.5


def kernel(feature, A, W1, b1, W2, b2, weight_b, weight_a):
    f_bf = feature.astype(jnp.bfloat16)
    w1_bf = W1.astype(jnp.bfloat16)
    w2_bf = W2.astype(jnp.bfloat16)
    b1_2d = b1.reshape(1, F)
    b2_2d = b2.reshape(1, F)

    _, out = pl.pallas_call(
        _body,
        grid=(2 * NB,),
        in_specs=[
            pl.BlockSpec(memory_space=pltpu.SMEM),
            pl.BlockSpec(memory_space=pltpu.SMEM),
            pl.BlockSpec((1, TM, N), lambda i: (0, jnp.minimum(i, NB - 1), 0)),
            pl.BlockSpec((1, TM, N), lambda i: (1, jnp.minimum(i, NB - 1), 0)),
            pl.BlockSpec((N, F), lambda i: (0, 0)),
            pl.BlockSpec((F, F), lambda i: (0, 0)),
            pl.BlockSpec((1, F), lambda i: (0, 0)),
            pl.BlockSpec((F, F), lambda i: (0, 0)),
            pl.BlockSpec((1, F), lambda i: (0, 0)),
        ],
        out_specs=[
            pl.BlockSpec(memory_space=pltpu.MemorySpace.HBM),
            pl.BlockSpec((TM, F), lambda i: (jnp.maximum(i - NB, 0), 0)),
        ],
        out_shape=[
            jax.ShapeDtypeStruct((N, N), jnp.float8_e4m3fn),
            jax.ShapeDtypeStruct((N, F), jnp.float32),
        ],
        scratch_shapes=[
            pltpu.VMEM((N, F), jnp.bfloat16),
            pltpu.VMEM((N, F), jnp.float32),
            pltpu.VMEM((N, F), jnp.float8_e4m3fn),
            pltpu.VMEM((2, TM, N), jnp.float8_e4m3fn),
            pltpu.VMEM((2, TM, N), jnp.float8_e4m3fn),
            pltpu.SemaphoreType.DMA((2,)),
            pltpu.SemaphoreType.DMA((2,)),
        ],
    )(weight_b, weight_a, A, A, f_bf, w1_bf, b1_2d, w2_bf, b2_2d)

    probe = _sc_merge(A, weight_b)
    return out + probe[:1, :1] * jnp.float32(1e-38)
